# Initial kernel scaffold; baseline (speedup 1.0000x reference)
import jax
import jax.numpy as jnp
from jax.experimental import pallas as pl


def kernel(z):
    raise NotImplementedError("write your pallas kernel here")


# speedup vs baseline: 33.6560x; 33.6560x over previous
import jax
import jax.numpy as jnp
from jax.experimental import pallas as pl

_B = 128
_N = 32768
_BISECT = 24


def _sparsemax_kernel(z_ref, out_ref):
    z = z_ref[:, :]
    row_max = jnp.max(z, axis=1, keepdims=True)
    lo = row_max - 1.0
    hi = row_max

    def body(_, carry):
        lo, hi = carry
        mid = 0.5 * (lo + hi)
        f = jnp.sum(jnp.maximum(z - mid, 0.0), axis=1, keepdims=True)
        pred = f >= 1.0
        lo = jnp.where(pred, mid, lo)
        hi = jnp.where(pred, hi, mid)
        return lo, hi

    lo, hi = jax.lax.fori_loop(0, _BISECT, body, (lo, hi))

    mask = z > lo
    k = jnp.sum(mask.astype(jnp.float32), axis=1, keepdims=True)
    s = jnp.sum(jnp.where(mask, z, 0.0), axis=1, keepdims=True)
    tau = (s - 1.0) / jnp.maximum(k, 1.0)

    p = jnp.maximum(z - tau, 0.0)
    col_mean = jnp.mean(p, axis=0, keepdims=True)
    out_ref[:, :] = jnp.broadcast_to(col_mean, (_B, _N))


def kernel(z):
    return pl.pallas_call(
        _sparsemax_kernel,
        out_shape=jax.ShapeDtypeStruct((_B, _N), z.dtype),
    )(z)


# speedup vs baseline: 39.2678x; 1.1667x over previous
import jax
import jax.numpy as jnp
from jax.experimental import pallas as pl

_B = 128
_N = 32768
_NEWTON = 12


def _sparsemax_kernel(z_ref, out_ref):
    z = z_ref[:, :]
    row_max = jnp.max(z, axis=1, keepdims=True)

    def body(_, t):
        mask = z > t
        k = jnp.sum(mask.astype(jnp.float32), axis=1, keepdims=True)
        s = jnp.sum(jnp.where(mask, z, 0.0), axis=1, keepdims=True)
        return (s - 1.0) / jnp.maximum(k, 1.0)

    tau = jax.lax.fori_loop(0, _NEWTON, body, row_max - 1.0)

    p = jnp.maximum(z - tau, 0.0)
    col_mean = jnp.mean(p, axis=0, keepdims=True)
    out_ref[:, :] = jnp.broadcast_to(col_mean, (_B, _N))


def kernel(z):
    return pl.pallas_call(
        _sparsemax_kernel,
        out_shape=jax.ShapeDtypeStruct((_B, _N), z.dtype),
    )(z)


# speedup vs baseline: 42.6913x; 1.0872x over previous
import jax
import jax.numpy as jnp
from jax.experimental import pallas as pl

_B = 128
_N = 32768
_NEWTON = 10


def _sparsemax_kernel(z_ref, out_ref):
    z = z_ref[:, :]
    row_max = jnp.max(z, axis=1, keepdims=True)

    def body(_, t):
        relu = jnp.maximum(z - t, 0.0)
        f = jnp.sum(relu, axis=1, keepdims=True)
        k = jnp.sum(jnp.where(relu > 0.0, 1.0, 0.0), axis=1, keepdims=True)
        return t + (f - 1.0) / jnp.maximum(k, 1.0)

    tau = jax.lax.fori_loop(0, _NEWTON, body, row_max - 1.0)

    p = jnp.maximum(z - tau, 0.0)
    col_mean = jnp.mean(p, axis=0, keepdims=True)
    out_ref[:, :] = jnp.broadcast_to(col_mean, (_B, _N))


def kernel(z):
    return pl.pallas_call(
        _sparsemax_kernel,
        out_shape=jax.ShapeDtypeStruct((_B, _N), z.dtype),
    )(z)


# speedup vs baseline: 44.0461x; 1.0317x over previous
import jax
import jax.numpy as jnp
from jax.experimental import pallas as pl

_B = 128
_N = 32768
_SECANT = 12


def _sparsemax_kernel(z_ref, out_ref):
    z = z_ref[:, :]
    row_max = jnp.max(z, axis=1, keepdims=True)

    t0 = row_max - 1.0
    relu = jnp.maximum(z - t0, 0.0)
    f0 = jnp.sum(relu, axis=1, keepdims=True)
    k0 = jnp.sum(jnp.where(relu > 0.0, 1.0, 0.0), axis=1, keepdims=True)
    t1 = t0 + (f0 - 1.0) / jnp.maximum(k0, 1.0)

    def body(_, carry):
        tp, fp, t = carry
        f = jnp.sum(jnp.maximum(z - t, 0.0), axis=1, keepdims=True)
        step = (f - 1.0) * (t - tp) / jnp.maximum(fp - f, 1e-30)
        step = jnp.clip(step, 0.0, jnp.maximum(f - 1.0, 0.0))
        return t, f, t + step

    _, _, tau = jax.lax.fori_loop(0, _SECANT, body, (t0, f0, t1))

    for _ in range(2):
        tau = jnp.clip(tau, row_max - 1.0, row_max - 1.0 / _N)
        mask = z > tau
        k = jnp.sum(jnp.where(mask, 1.0, 0.0), axis=1, keepdims=True)
        s = jnp.sum(jnp.where(mask, z, 0.0), axis=1, keepdims=True)
        tau = (s - 1.0) / jnp.maximum(k, 1.0)

    p = jnp.maximum(z - tau, 0.0)
    col_mean = jnp.mean(p, axis=0, keepdims=True)
    out_ref[:, :] = jnp.broadcast_to(col_mean, (_B, _N))


def kernel(z):
    return pl.pallas_call(
        _sparsemax_kernel,
        out_shape=jax.ShapeDtypeStruct((_B, _N), z.dtype),
    )(z)
